# Initial kernel scaffold; baseline (speedup 1.0000x reference)
import functools

import jax
import jax.numpy as jnp
from jax import lax
from jax.experimental import pallas as pl
from jax.experimental.pallas import tpu as pltpu
from jax.experimental.pallas import tpu_sc as plsc

_N = 10000
_E = 320000
_D = 128
_NC = 2
_NS = 16
_NW = _NC * _NS
_K = 80
_CPW = _E // (_NW * _K)
_ZR = 125


def _mm_body(x_ref, w_ref, o_ref):
    o_ref[...] = jnp.dot(x_ref[...], w_ref[...],
                         preferred_element_type=jnp.float32)


def _matmul(x, W):
    return pl.pallas_call(
        _mm_body,
        grid=(10,),
        in_specs=[
            pl.BlockSpec((_N // 10, _D), lambda i: (i, 0)),
            pl.BlockSpec((_D, _D), lambda i: (0, 0)),
        ],
        out_specs=pl.BlockSpec((_N // 10, _D), lambda i: (i, 0)),
        out_shape=jax.ShapeDtypeStruct((_N, _D), jnp.float32),
    )(x, W)


_sc_mesh = plsc.VectorSubcoreMesh(core_axis_name="c", subcore_axis_name="s")


@functools.partial(
    pl.kernel,
    out_type=jax.ShapeDtypeStruct((_NC, _N, _D), jnp.float32),
    mesh=_sc_mesh,
    scratch_types=[
        pltpu.VMEM((_CPW, _K), jnp.int32),
        pltpu.VMEM((_CPW, _K), jnp.int32),
        pltpu.VMEM((_K, _D), jnp.float32),
        pltpu.VMEM((_K, _D), jnp.float32),
        pltpu.VMEM((_ZR, _D), jnp.float32),
        pltpu.VMEM_SHARED((_N, _D), jnp.float32),
        pltpu.SemaphoreType.DMA,
        pltpu.SemaphoreType.DMA,
    ],
)
def _sc_agg(src_hbm, dst_hbm, xw_hbm, zrow_hbm, out_hbm,
            src_v, dst_v, rows_a, rows_b, zb, acc, sem_a, sem_b):
    cid = lax.axis_index("c")
    sid = lax.axis_index("s")
    w = cid * _NS + sid

    pltpu.sync_copy(zrow_hbm, zb)

    def _zero(i, carry):
        pltpu.sync_copy(zb, acc.at[pl.ds((sid * 5 + i) * _ZR, _ZR)])
        return carry

    lax.fori_loop(0, 5, _zero, 0)

    pltpu.sync_copy(src_hbm.at[pl.ds(w * _CPW, _CPW)], src_v)
    pltpu.sync_copy(dst_hbm.at[pl.ds(w * _CPW, _CPW)], dst_v)
    plsc.subcore_barrier()

    def _gather(c, rows, sem):
        pltpu.async_copy(xw_hbm.at[src_v.at[c]], rows, sem)

    def _gwait(c, rows, sem):
        pltpu.make_async_copy(xw_hbm.at[src_v.at[c]], rows, sem).wait()

    def _scat(c, rows):
        pltpu.sync_copy(rows, acc.at[dst_v.at[c]], add=True)

    _gather(0, rows_a, sem_a)
    _gather(1, rows_b, sem_b)

    def _body(i, carry):
        j = 2 * i
        _gwait(j, rows_a, sem_a)
        _scat(j, rows_a)
        _gather(j + 2, rows_a, sem_a)
        _gwait(j + 1, rows_b, sem_b)
        _scat(j + 1, rows_b)
        _gather(j + 3, rows_b, sem_b)
        return carry

    lax.fori_loop(0, (_CPW - 3) // 2, _body, 0)

    _gwait(_CPW - 3, rows_a, sem_a)
    _scat(_CPW - 3, rows_a)
    _gather(_CPW - 1, rows_a, sem_a)
    _gwait(_CPW - 2, rows_b, sem_b)
    _scat(_CPW - 2, rows_b)
    _gwait(_CPW - 1, rows_a, sem_a)
    _scat(_CPW - 1, rows_a)

    plsc.subcore_barrier()
    rpt = _N // _NS
    pltpu.sync_copy(acc.at[pl.ds(sid * rpt, rpt)],
                    out_hbm.at[cid, pl.ds(sid * rpt, rpt)])


def _cb_body(p_ref, o_ref):
    o_ref[...] = jnp.maximum(p_ref[0] + p_ref[1], 0.0)


def _combine(partials):
    return pl.pallas_call(
        _cb_body,
        grid=(10,),
        in_specs=[pl.BlockSpec((_NC, _N // 10, _D), lambda i: (0, i, 0))],
        out_specs=pl.BlockSpec((_N // 10, _D), lambda i: (i, 0)),
        out_shape=jax.ShapeDtypeStruct((_N, _D), jnp.float32),
    )(partials)


def kernel(x, edge_index, W):
    xw = _matmul(x, W)
    ei = edge_index.astype(jnp.int32)
    src = ei[0].reshape(_NW * _CPW, _K)
    dst = ei[1].reshape(_NW * _CPW, _K)
    zrow = jnp.zeros((_ZR, _D), jnp.float32)
    partials = _sc_agg(src, dst, xw, zrow)
    return _combine(partials)


# speedup vs baseline: 10.3491x; 10.3491x over previous
import functools

import jax
import jax.numpy as jnp
from jax import lax
from jax.experimental import pallas as pl
from jax.experimental.pallas import tpu as pltpu
from jax.experimental.pallas import tpu_sc as plsc

_N = 10000
_NP = 10240
_E = 320000
_D = 128
_NC = 2
_NS = 16
_NW = _NC * _NS
_K = 80
_CPW = _E // (_NW * _K)


def _mm_body(x_ref, w_ref, o_ref):
    o_ref[...] = jnp.dot(x_ref[...], w_ref[...],
                         preferred_element_type=jnp.float32)


def _matmul(x, W):
    return pl.pallas_call(
        _mm_body,
        grid=(10,),
        in_specs=[
            pl.BlockSpec((_N // 10, _D), lambda i: (i, 0)),
            pl.BlockSpec((_D, _D), lambda i: (0, 0)),
        ],
        out_specs=pl.BlockSpec((_N // 10, _D), lambda i: (i, 0)),
        out_shape=jax.ShapeDtypeStruct((_N, _D), jnp.float32),
    )(x, W)


_sc_mesh = plsc.VectorSubcoreMesh(core_axis_name="c", subcore_axis_name="s")


@functools.partial(
    pl.kernel,
    out_type=jax.ShapeDtypeStruct((_NC, _NP, _D), jnp.float32),
    mesh=_sc_mesh,
    scratch_types=[
        pltpu.VMEM((2, _K), jnp.int32),
        pltpu.VMEM((2, _K), jnp.int32),
        pltpu.VMEM((_K, _D), jnp.float32),
        pltpu.VMEM((_K, _D), jnp.float32),
        pltpu.VMEM_SHARED((_NP, _D), jnp.float32),
        pltpu.SemaphoreType.DMA,
        pltpu.SemaphoreType.DMA,
        pltpu.SemaphoreType.DMA,
        pltpu.SemaphoreType.DMA,
    ],
)
def _sc_agg(idx_hbm, xw_hbm, zrow_hbm, out_hbm,
            idx_a, idx_b, rows_a, rows_b, acc,
            si_a, si_b, sg_a, sg_b):
    cid = lax.axis_index("c")
    sid = lax.axis_index("s")
    w = cid * _NS + sid

    pltpu.sync_copy(zrow_hbm, acc.at[pl.ds(sid * 640, 640)])

    def _iload(c, ibuf, sem):
        pltpu.async_copy(idx_hbm.at[w, c], ibuf, sem)

    def _iwait(c, ibuf, sem):
        pltpu.make_async_copy(idx_hbm.at[w, c], ibuf, sem).wait()

    def _gather(ibuf, rows, sem):
        pltpu.async_copy(xw_hbm.at[ibuf.at[0]], rows, sem)

    def _gwait(ibuf, rows, sem):
        pltpu.make_async_copy(xw_hbm.at[ibuf.at[0]], rows, sem).wait()

    def _scat(ibuf, rows):
        pltpu.sync_copy(rows, acc.at[ibuf.at[1]], add=True)

    A = (idx_a, rows_a, si_a, sg_a)
    B = (idx_b, rows_b, si_b, sg_b)

    def _step(j, cur, nxt, gather_next=True, load_next2=True):
        ci, cr, csi, csg = cur
        ni, nr, nsi, nsg = nxt
        if gather_next:
            _iwait(j + 1, ni, nsi)
            _gather(ni, nr, nsg)
        _gwait(ci, cr, csg)
        _scat(ci, cr)
        if load_next2:
            _iload(j + 2, ci, csi)

    plsc.subcore_barrier()

    _iload(0, idx_a, si_a)
    _iload(1, idx_b, si_b)
    _iwait(0, idx_a, si_a)
    _gather(idx_a, rows_a, sg_a)

    def _body(i, carry):
        j = 2 * i
        _step(j, A, B)
        _step(j + 1, B, A)
        return carry

    lax.fori_loop(0, (_CPW - 3) // 2, _body, 0)

    _step(_CPW - 3, A, B, load_next2=True)
    _step(_CPW - 2, B, A, gather_next=True, load_next2=False)
    _step(_CPW - 1, A, B, gather_next=False, load_next2=False)

    plsc.subcore_barrier()
    pltpu.sync_copy(acc.at[pl.ds(sid * 640, 640)],
                    out_hbm.at[cid, pl.ds(sid * 640, 640)])


def _cb_body(p_ref, o_ref):
    o_ref[...] = jnp.maximum(p_ref[0] + p_ref[1], 0.0)


def _combine(partials):
    return pl.pallas_call(
        _cb_body,
        grid=(10,),
        in_specs=[pl.BlockSpec((_NC, _NP // 10, _D), lambda i: (0, i, 0))],
        out_specs=pl.BlockSpec((_NP // 10, _D), lambda i: (i, 0)),
        out_shape=jax.ShapeDtypeStruct((_NP, _D), jnp.float32),
    )(partials)


def kernel(x, edge_index, W):
    xw = _matmul(x, W)
    ei = edge_index.astype(jnp.int32)
    idx = jnp.stack(
        [ei[0].reshape(_NW, _CPW, _K), ei[1].reshape(_NW, _CPW, _K)], axis=2)
    zrow = jnp.zeros((640, _D), jnp.float32)
    partials = _sc_agg(idx, xw, zrow)
    return _combine(partials)[:_N]


# speedup vs baseline: 10.8470x; 1.0481x over previous
import functools

import jax
import jax.numpy as jnp
from jax import lax
from jax.experimental import pallas as pl
from jax.experimental.pallas import tpu as pltpu
from jax.experimental.pallas import tpu_sc as plsc

_N = 10000
_NP = 10240
_E = 320000
_D = 128
_NC = 2
_NS = 16
_NW = _NC * _NS
_K = 100
_CPW = _E // (_NW * _K)


def _mm_body(x_ref, w_ref, o_ref):
    o_ref[...] = jnp.dot(x_ref[...], w_ref[...],
                         preferred_element_type=jnp.float32)


def _matmul(x, W):
    return pl.pallas_call(
        _mm_body,
        grid=(10,),
        in_specs=[
            pl.BlockSpec((_N // 10, _D), lambda i: (i, 0)),
            pl.BlockSpec((_D, _D), lambda i: (0, 0)),
        ],
        out_specs=pl.BlockSpec((_N // 10, _D), lambda i: (i, 0)),
        out_shape=jax.ShapeDtypeStruct((_N, _D), jnp.float32),
    )(x, W)


_sc_mesh = plsc.VectorSubcoreMesh(core_axis_name="c", subcore_axis_name="s")


@functools.partial(
    pl.kernel,
    out_type=jax.ShapeDtypeStruct((_NC, _NP, _D), jnp.float32),
    mesh=_sc_mesh,
    scratch_types=[
        pltpu.VMEM((2, 1, _K), jnp.int32),
        pltpu.VMEM((2, 1, _K), jnp.int32),
        pltpu.VMEM((_K, _D), jnp.float32),
        pltpu.VMEM((_K, _D), jnp.float32),
        pltpu.VMEM_SHARED((_NP, _D), jnp.float32),
        pltpu.SemaphoreType.DMA,
        pltpu.SemaphoreType.DMA,
        pltpu.SemaphoreType.DMA,
        pltpu.SemaphoreType.DMA,
    ],
)
def _sc_agg(idx_hbm, xw_hbm, zrow_hbm, out_hbm,
            idx_a, idx_b, rows_a, rows_b, acc,
            si_a, si_b, sg_a, sg_b):
    cid = lax.axis_index("c")
    sid = lax.axis_index("s")
    w = cid * _NS + sid

    pltpu.sync_copy(zrow_hbm, acc.at[pl.ds(sid * 640, 640)])

    def _iload(c, ibuf, sem):
        g = w * _CPW + c
        pltpu.async_copy(idx_hbm.at[0, g], ibuf.at[0], sem)
        pltpu.async_copy(idx_hbm.at[1, g], ibuf.at[1], sem)

    def _iwait(c, ibuf, sem):
        g = w * _CPW + c
        pltpu.make_async_copy(idx_hbm.at[0, g], ibuf.at[0], sem).wait()
        pltpu.make_async_copy(idx_hbm.at[1, g], ibuf.at[1], sem).wait()

    def _gather(ibuf, rows, sem):
        pltpu.async_copy(xw_hbm.at[ibuf.at[0, 0]], rows, sem)

    def _gwait(ibuf, rows, sem):
        pltpu.make_async_copy(xw_hbm.at[ibuf.at[0, 0]], rows, sem).wait()

    def _scat(ibuf, rows):
        pltpu.sync_copy(rows, acc.at[ibuf.at[1, 0]], add=True)

    A = (idx_a, rows_a, si_a, sg_a)
    B = (idx_b, rows_b, si_b, sg_b)

    def _step(j, cur, nxt, gather_next=True, load_next2=True):
        ci, cr, csi, csg = cur
        ni, nr, nsi, nsg = nxt
        if gather_next:
            _iwait(j + 1, ni, nsi)
            _gather(ni, nr, nsg)
        _gwait(ci, cr, csg)
        _scat(ci, cr)
        if load_next2:
            _iload(j + 2, ci, csi)

    plsc.subcore_barrier()

    _iload(0, idx_a, si_a)
    _iload(1, idx_b, si_b)
    _iwait(0, idx_a, si_a)
    _gather(idx_a, rows_a, sg_a)

    def _body(i, carry):
        j = 2 * i
        _step(j, A, B)
        _step(j + 1, B, A)
        return carry

    lax.fori_loop(0, (_CPW - 2) // 2, _body, 0)

    _step(_CPW - 2, A, B, gather_next=True, load_next2=False)
    _step(_CPW - 1, B, A, gather_next=False, load_next2=False)

    plsc.subcore_barrier()
    pltpu.sync_copy(acc.at[pl.ds(sid * 640, 640)],
                    out_hbm.at[cid, pl.ds(sid * 640, 640)])


def _cb_body(p_ref, o_ref):
    o_ref[...] = jnp.maximum(p_ref[0] + p_ref[1], 0.0)


def _combine(partials):
    return pl.pallas_call(
        _cb_body,
        grid=(10,),
        in_specs=[pl.BlockSpec((_NC, _N // 10, _D), lambda i: (0, i, 0))],
        out_specs=pl.BlockSpec((_N // 10, _D), lambda i: (i, 0)),
        out_shape=jax.ShapeDtypeStruct((_N, _D), jnp.float32),
    )(partials)


def kernel(x, edge_index, W):
    xw = _matmul(x, W)
    idx = edge_index.astype(jnp.int32).reshape(2, _NW * _CPW, 1, _K)
    zrow = jnp.zeros((640, _D), jnp.float32)
    partials = _sc_agg(idx, xw, zrow)
    return _combine(partials)
